# initial kernel scaffold (unmeasured)
import jax
import jax.numpy as jnp
from jax import lax
from jax.experimental import pallas as pl
from jax.experimental.pallas import tpu as pltpu


def kernel(
    x,
):
    def body(*refs):
        pass

    out_shape = jax.ShapeDtypeStruct(..., jnp.float32)
    return pl.pallas_call(body, out_shape=out_shape)(...)


# baseline (device time: 29103 ns/iter reference)
import jax
import jax.numpy as jnp
from jax import lax
from jax.experimental import pallas as pl
from jax.experimental.pallas import tpu as pltpu

N_Z = 4
M = 512
N_PER = 512


def kernel(x):
    _, m, n_total = x.shape
    assert m == M and n_total == N_PER * N_Z

    def body(x_ref, out_ref, send_bufs, recv_bufs, send_sems, recv_sems):
        my_x = lax.axis_index("x")
        my_y = lax.axis_index("y")
        my_z = lax.axis_index("z")
        left = lax.rem(my_z - 1 + N_Z, N_Z)
        right = lax.rem(my_z + 1, N_Z)

        barrier_sem = pltpu.get_barrier_semaphore()
        for nbr in [left, right]:
            pl.semaphore_signal(
                barrier_sem, inc=1,
                device_id=(my_x, my_y, nbr),
                device_id_type=pl.DeviceIdType.MESH,
            )
        pl.semaphore_wait(barrier_sem, 2)

        def chunk(j):
            return x_ref[0, :, pl.ds(j * N_PER, N_PER)].astype(jnp.bfloat16)

        for s in range(N_Z - 1):
            j_send = lax.rem(my_z - s - 1 + 2 * N_Z, N_Z)
            j_recv = lax.rem(my_z - s - 2 + 2 * N_Z, N_Z)

            if s == 0:
                send_bufs[0, :, :] = chunk(j_send)

            rdma = pltpu.make_async_remote_copy(
                src_ref=send_bufs.at[s],
                dst_ref=recv_bufs.at[s],
                send_sem=send_sems.at[s],
                recv_sem=recv_sems.at[s],
                device_id=(my_x, my_y, right),
                device_id_type=pl.DeviceIdType.MESH,
            )
            rdma.start()
            rdma.wait()

            partial = recv_bufs[s, :, :].astype(jnp.float32) + x_ref[
                0, :, pl.ds(j_recv * N_PER, N_PER)
            ]
            if s < N_Z - 2:
                send_bufs[s + 1, :, :] = partial.astype(jnp.bfloat16)
            else:
                out_ref[:, :] = partial

    return pl.pallas_call(
        body,
        out_shape=jax.ShapeDtypeStruct((M, N_PER), jnp.float32),
        in_specs=[pl.BlockSpec(memory_space=pltpu.VMEM)],
        out_specs=pl.BlockSpec(memory_space=pltpu.VMEM),
        scratch_shapes=[
            pltpu.VMEM((N_Z - 1, M, N_PER), jnp.bfloat16),
            pltpu.VMEM((N_Z - 1, M, N_PER), jnp.bfloat16),
            pltpu.SemaphoreType.DMA((N_Z - 1,)),
            pltpu.SemaphoreType.DMA((N_Z - 1,)),
        ],
        compiler_params=pltpu.CompilerParams(collective_id=0),
    )(x)


# device time: 28562 ns/iter; 1.0189x vs baseline; 1.0189x over previous
import jax
import jax.numpy as jnp
from jax import lax
from jax.experimental import pallas as pl
from jax.experimental.pallas import tpu as pltpu

N_Z = 4
M = 512
NP = 512
S = 1
MP = M // S
N_R = N_Z - 1


def kernel(x):
    _, m, n_total = x.shape
    assert m == M and n_total == NP * N_Z

    def body(x_ref, out_ref, sbr, sbl, rbr, rbl, ssr, ssl, rsr, rsl):
        my_x = lax.axis_index("x")
        my_y = lax.axis_index("y")
        my_z = lax.axis_index("z")
        zr = lax.min(my_z + 1, N_Z - 1)
        zl = lax.max(my_z - 1, 0)

        bsem = pltpu.get_barrier_semaphore()
        for nbr in [zl, zr]:
            pl.semaphore_signal(
                bsem, inc=1,
                device_id=(my_x, my_y, nbr),
                device_id_type=pl.DeviceIdType.MESH,
            )
        pl.semaphore_wait(bsem, 2)

        def cpiece(j, p):
            return x_ref[0, pl.ds(p * MP, MP), pl.ds(j * NP, NP)]

        def mk(right, r, p):
            if right:
                sb, rb, ss, rs, tz = sbr, rbr, ssr, rsr, zr
            else:
                sb, rb, ss, rs, tz = sbl, rbl, ssl, rsl, zl
            return pltpu.make_async_remote_copy(
                src_ref=sb.at[r, p],
                dst_ref=rb.at[r, p],
                send_sem=ss.at[r, p],
                recv_sem=rs.at[r, p],
                device_id=(my_x, my_y, tz),
                device_id_type=pl.DeviceIdType.MESH,
            )

        for r in range(N_R):
            send_right = my_z <= r
            send_left = my_z >= N_Z - 1 - r
            jr = lax.min(N_Z - 1 - r + my_z, N_Z - 1)
            jl = lax.max(r + my_z - (N_Z - 1), 0)
            for p in range(S):
                @pl.when(send_right)
                def _(r=r, p=p, jr=jr):
                    val = cpiece(jr, p)
                    if r > 0:
                        val = val + jnp.where(
                            my_z >= 1, rbr[r - 1, p].astype(jnp.float32), 0.0
                        )
                    sbr[r, p] = val.astype(jnp.bfloat16)
                    mk(True, r, p).start()

                @pl.when(send_left)
                def _(r=r, p=p, jl=jl):
                    val = cpiece(jl, p)
                    if r > 0:
                        val = val + jnp.where(
                            my_z <= N_Z - 2, rbl[r - 1, p].astype(jnp.float32), 0.0
                        )
                    sbl[r, p] = val.astype(jnp.bfloat16)
                    mk(False, r, p).start()

            recv_right = (my_z >= 1) & (my_z <= r + 1)
            recv_left = (my_z <= N_Z - 2) & (my_z >= N_Z - 2 - r)
            for p in range(S):
                @pl.when(recv_right)
                def _(r=r, p=p):
                    mk(True, r, p).wait_recv()

                @pl.when(recv_left)
                def _(r=r, p=p):
                    mk(False, r, p).wait_recv()

        for p in range(S):
            val = cpiece(my_z, p)
            val = val + jnp.where(
                my_z >= 1, rbr[N_R - 1, p].astype(jnp.float32), 0.0
            )
            val = val + jnp.where(
                my_z <= N_Z - 2, rbl[N_R - 1, p].astype(jnp.float32), 0.0
            )
            out_ref[pl.ds(p * MP, MP), :] = val

        for r in range(N_R):
            for p in range(S):
                @pl.when(my_z <= r)
                def _(r=r, p=p):
                    mk(True, r, p).wait_send()

                @pl.when(my_z >= N_Z - 1 - r)
                def _(r=r, p=p):
                    mk(False, r, p).wait_send()

    buf = pltpu.VMEM((N_R, S, MP, NP), jnp.bfloat16)
    sem = pltpu.SemaphoreType.DMA((N_R, S))
    return pl.pallas_call(
        body,
        out_shape=jax.ShapeDtypeStruct((M, NP), jnp.float32),
        in_specs=[pl.BlockSpec(memory_space=pltpu.VMEM)],
        out_specs=pl.BlockSpec(memory_space=pltpu.VMEM),
        scratch_shapes=[buf, buf, buf, buf, sem, sem, sem, sem],
        compiler_params=pltpu.CompilerParams(collective_id=0),
    )(x)


# device time: 23354 ns/iter; 1.2462x vs baseline; 1.2230x over previous
import jax
import jax.numpy as jnp
from jax import lax
from jax.experimental import pallas as pl
from jax.experimental.pallas import tpu as pltpu

N_Z = 4
M = 512
NP = 512
S = 2
MP = M // S
N_R = N_Z - 1


def kernel(x):
    _, m, n_total = x.shape
    assert m == M and n_total == NP * N_Z

    def body(x_ref, out_ref, sbr, sbl, rbr, rbl, ssr, ssl, rsr, rsl):
        my_x = lax.axis_index("x")
        my_y = lax.axis_index("y")
        my_z = lax.axis_index("z")
        zr = lax.min(my_z + 1, N_Z - 1)
        zl = lax.max(my_z - 1, 0)

        bsem = pltpu.get_barrier_semaphore()
        for nbr in [zl, zr]:
            pl.semaphore_signal(
                bsem, inc=1,
                device_id=(my_x, my_y, nbr),
                device_id_type=pl.DeviceIdType.MESH,
            )
        pl.semaphore_wait(bsem, 2)

        def cpiece(j, p):
            return x_ref[0, pl.ds(p * MP, MP), pl.ds(j * NP, NP)].astype(
                jnp.bfloat16
            )

        def mk(right, r, p):
            if right:
                sb, rb, ss, rs, tz = sbr, rbr, ssr, rsr, zr
            else:
                sb, rb, ss, rs, tz = sbl, rbl, ssl, rsl, zl
            return pltpu.make_async_remote_copy(
                src_ref=sb.at[r, p],
                dst_ref=rb.at[r, p],
                send_sem=ss.at[r, p],
                recv_sem=rs.at[r, p],
                device_id=(my_x, my_y, tz),
                device_id_type=pl.DeviceIdType.MESH,
            )

        for r in range(N_R):
            send_right = my_z <= r
            send_left = my_z >= N_Z - 1 - r
            jr = lax.min(N_Z - 1 - r + my_z, N_Z - 1)
            jl = lax.max(r + my_z - (N_Z - 1), 0)
            for p in range(S):
                if r > 0:
                    @pl.when(send_right & (my_z >= 1))
                    def _(r=r, p=p):
                        mk(True, r - 1, p).wait_recv()

                @pl.when(send_right)
                def _(r=r, p=p, jr=jr):
                    val = cpiece(jr, p)
                    if r > 0:
                        val = val + jnp.where(
                            my_z >= 1, rbr[r - 1, p], jnp.bfloat16(0)
                        )
                    sbr[r, p] = val
                    mk(True, r, p).start()

                if r > 0:
                    @pl.when(send_left & (my_z <= N_Z - 2))
                    def _(r=r, p=p):
                        mk(False, r - 1, p).wait_recv()

                @pl.when(send_left)
                def _(r=r, p=p, jl=jl):
                    val = cpiece(jl, p)
                    if r > 0:
                        val = val + jnp.where(
                            my_z <= N_Z - 2, rbl[r - 1, p], jnp.bfloat16(0)
                        )
                    sbl[r, p] = val
                    mk(False, r, p).start()

        for p in range(S):
            @pl.when(my_z >= 1)
            def _(p=p):
                mk(True, N_R - 1, p).wait_recv()

            @pl.when(my_z <= N_Z - 2)
            def _(p=p):
                mk(False, N_R - 1, p).wait_recv()

            val = x_ref[0, pl.ds(p * MP, MP), pl.ds(my_z * NP, NP)]
            val = val + jnp.where(
                my_z >= 1, rbr[N_R - 1, p].astype(jnp.float32), 0.0
            )
            val = val + jnp.where(
                my_z <= N_Z - 2, rbl[N_R - 1, p].astype(jnp.float32), 0.0
            )
            out_ref[pl.ds(p * MP, MP), :] = val

        for r in range(N_R):
            for p in range(S):
                @pl.when(my_z <= r)
                def _(r=r, p=p):
                    mk(True, r, p).wait_send()

                @pl.when(my_z >= N_Z - 1 - r)
                def _(r=r, p=p):
                    mk(False, r, p).wait_send()

    buf = pltpu.VMEM((N_R, S, MP, NP), jnp.bfloat16)
    sem = pltpu.SemaphoreType.DMA((N_R, S))
    return pl.pallas_call(
        body,
        out_shape=jax.ShapeDtypeStruct((M, NP), jnp.float32),
        in_specs=[pl.BlockSpec(memory_space=pltpu.VMEM)],
        out_specs=pl.BlockSpec(memory_space=pltpu.VMEM),
        scratch_shapes=[buf, buf, buf, buf, sem, sem, sem, sem],
        compiler_params=pltpu.CompilerParams(collective_id=0),
    )(x)


# device time: 23267 ns/iter; 1.2508x vs baseline; 1.0037x over previous
import jax
import jax.numpy as jnp
from jax import lax
from jax.experimental import pallas as pl
from jax.experimental.pallas import tpu as pltpu

N_Z = 4
M = 512
NP = 512
S = 4
MP = M // S
N_R = N_Z - 1


def kernel(x):
    _, m, n_total = x.shape
    assert m == M and n_total == NP * N_Z

    def body(x_ref, out_ref, sbr, sbl, rbr, rbl, ssr, ssl, rsr, rsl):
        my_x = lax.axis_index("x")
        my_y = lax.axis_index("y")
        my_z = lax.axis_index("z")
        zr = lax.min(my_z + 1, N_Z - 1)
        zl = lax.max(my_z - 1, 0)

        bsem = pltpu.get_barrier_semaphore()
        for nbr in [zl, zr]:
            pl.semaphore_signal(
                bsem, inc=1,
                device_id=(my_x, my_y, nbr),
                device_id_type=pl.DeviceIdType.MESH,
            )
        pl.semaphore_wait(bsem, 2)

        def cpiece(j, p):
            return x_ref[0, pl.ds(p * MP, MP), pl.ds(j * NP, NP)].astype(
                jnp.bfloat16
            )

        def mk(right, r, p):
            if right:
                sb, rb, ss, rs, tz = sbr, rbr, ssr, rsr, zr
            else:
                sb, rb, ss, rs, tz = sbl, rbl, ssl, rsl, zl
            return pltpu.make_async_remote_copy(
                src_ref=sb.at[r, p],
                dst_ref=rb.at[r, p],
                send_sem=ss.at[r, p],
                recv_sem=rs.at[r, p],
                device_id=(my_x, my_y, tz),
                device_id_type=pl.DeviceIdType.MESH,
            )

        for r in range(N_R):
            send_right = my_z <= r
            send_left = my_z >= N_Z - 1 - r
            jr = lax.min(N_Z - 1 - r + my_z, N_Z - 1)
            jl = lax.max(r + my_z - (N_Z - 1), 0)
            for p in range(S):
                if r > 0:
                    @pl.when(send_right & (my_z >= 1))
                    def _(r=r, p=p):
                        mk(True, r - 1, p).wait_recv()

                @pl.when(send_right)
                def _(r=r, p=p, jr=jr):
                    val = cpiece(jr, p)
                    if r > 0:
                        val = val + jnp.where(
                            my_z >= 1, rbr[r - 1, p], jnp.bfloat16(0)
                        )
                    sbr[r, p] = val
                    mk(True, r, p).start()

                if r > 0:
                    @pl.when(send_left & (my_z <= N_Z - 2))
                    def _(r=r, p=p):
                        mk(False, r - 1, p).wait_recv()

                @pl.when(send_left)
                def _(r=r, p=p, jl=jl):
                    val = cpiece(jl, p)
                    if r > 0:
                        val = val + jnp.where(
                            my_z <= N_Z - 2, rbl[r - 1, p], jnp.bfloat16(0)
                        )
                    sbl[r, p] = val
                    mk(False, r, p).start()

        for p in range(S):
            @pl.when(my_z >= 1)
            def _(p=p):
                mk(True, N_R - 1, p).wait_recv()

            @pl.when(my_z <= N_Z - 2)
            def _(p=p):
                mk(False, N_R - 1, p).wait_recv()

            val = x_ref[0, pl.ds(p * MP, MP), pl.ds(my_z * NP, NP)]
            val = val + jnp.where(
                my_z >= 1, rbr[N_R - 1, p].astype(jnp.float32), 0.0
            )
            val = val + jnp.where(
                my_z <= N_Z - 2, rbl[N_R - 1, p].astype(jnp.float32), 0.0
            )
            out_ref[pl.ds(p * MP, MP), :] = val

        for r in range(N_R):
            for p in range(S):
                @pl.when(my_z <= r)
                def _(r=r, p=p):
                    mk(True, r, p).wait_send()

                @pl.when(my_z >= N_Z - 1 - r)
                def _(r=r, p=p):
                    mk(False, r, p).wait_send()

    buf = pltpu.VMEM((N_R, S, MP, NP), jnp.bfloat16)
    sem = pltpu.SemaphoreType.DMA((N_R, S))
    return pl.pallas_call(
        body,
        out_shape=jax.ShapeDtypeStruct((M, NP), jnp.float32),
        in_specs=[pl.BlockSpec(memory_space=pltpu.VMEM)],
        out_specs=pl.BlockSpec(memory_space=pltpu.VMEM),
        scratch_shapes=[buf, buf, buf, buf, sem, sem, sem, sem],
        compiler_params=pltpu.CompilerParams(collective_id=0),
    )(x)
